# baseline (speedup 1.0000x reference)
import jax
import jax.numpy as jnp
from jax import lax
from jax.experimental import pallas as pl
from jax.experimental.pallas import tpu as pltpu
from jax.experimental.pallas import tpu_sc as plsc

_N = 5000
_L = 16
_NW = 32
_RPW = 160
_NP = _NW * _RPW
_RU = 4
_THR = 0.4


def _decay_body(x0h, y0h, x1h, y1h, sch, outh,
                x0v, y0v, x1v, y1v, scv, arv, outv):
    cid = lax.axis_index("c")
    sid = lax.axis_index("s")
    wid = sid * 2 + cid
    base = wid * _RPW

    pltpu.sync_copy(x0h, x0v)
    pltpu.sync_copy(y0h, y0v)
    pltpu.sync_copy(x1h, x1v)
    pltpu.sync_copy(y1h, y1v)
    pltpu.sync_copy(sch, scv)

    def area_chunk(k, carry):
        sl = pl.ds(k * _L, _L)
        arv[sl] = (x1v[sl] - x0v[sl]) * (y1v[sl] - y0v[sl])
        return carry
    lax.fori_loop(0, _NP // _L, area_chunk, 0)

    lanes = lax.iota(jnp.int32, _L)
    ones = jnp.full((_L,), 1.0, jnp.float32)

    def row_block(rb, carry):
        i0 = base + rb * _L
        sl_i = pl.ds(i0, _L)
        rx0 = x0v[sl_i]
        ry0 = y0v[sl_i]
        rx1 = x1v[sl_i]
        ry1 = y1v[sl_i]
        rar = arv[sl_i]
        rsc = scv[sl_i]

        res = jnp.zeros((_L,), jnp.float32)
        for g in range(_L // _RU):
            ix0 = [rx0[g * _RU + r] for r in range(_RU)]
            iy0 = [ry0[g * _RU + r] for r in range(_RU)]
            ix1 = [rx1[g * _RU + r] for r in range(_RU)]
            iy1 = [ry1[g * _RU + r] for r in range(_RU)]
            iar = [rar[g * _RU + r] for r in range(_RU)]
            isc = [rsc[g * _RU + r] for r in range(_RU)]

            def jchunk(k, accs):
                sl = pl.ds(k * _L, _L)
                jx0 = x0v[sl]
                jy0 = y0v[sl]
                jx1 = x1v[sl]
                jy1 = y1v[sl]
                js = scv[sl]
                ja = arv[sl]
                nxt = []
                for r in range(_RU):
                    wx = jnp.maximum(jnp.minimum(ix1[r], jx1) - jnp.maximum(ix0[r], jx0), 0.0)
                    wy = jnp.maximum(jnp.minimum(iy1[r], jy1) - jnp.maximum(iy0[r], jy0), 0.0)
                    inter = wx * wy
                    union = iar[r] + ja - inter
                    hit = (inter > _THR * union) & (js > isc[r])
                    ratio = (union - inter) / union
                    nxt.append(accs[r] * jnp.where(hit, ratio, 1.0))
                return tuple(nxt)

            accs = lax.fori_loop(0, _NP // _L, jchunk, (ones,) * _RU)
            for r in range(_RU):
                a = accs[r]
                for s in (1, 2, 4, 8):
                    a = a * a.at[lanes ^ s].get(mode="promise_in_bounds")
                res = jnp.where(lanes == (g * _RU + r), a[0], res)
        outv[pl.ds(rb * _L, _L)] = res
        return carry

    lax.fori_loop(0, _RPW // _L, row_block, 0)
    pltpu.sync_copy(outv, outh.at[pl.ds(base, _RPW)])


_mesh = plsc.VectorSubcoreMesh(core_axis_name="c", subcore_axis_name="s")

_decay_call = pl.kernel(
    _decay_body,
    out_type=jax.ShapeDtypeStruct((_NP,), jnp.float32),
    mesh=_mesh,
    scratch_types=[
        pltpu.VMEM((_NP,), jnp.float32),
        pltpu.VMEM((_NP,), jnp.float32),
        pltpu.VMEM((_NP,), jnp.float32),
        pltpu.VMEM((_NP,), jnp.float32),
        pltpu.VMEM((_NP,), jnp.float32),
        pltpu.VMEM((_NP,), jnp.float32),
        pltpu.VMEM((_RPW,), jnp.float32),
    ],
)


def kernel(boxes, scores):
    pad = _NP - _N
    big = jnp.float32(4.0e8)
    x0 = jnp.concatenate([boxes[:, 0], jnp.full((pad,), big, jnp.float32)])
    y0 = jnp.concatenate([boxes[:, 1], jnp.full((pad,), big, jnp.float32)])
    x1 = jnp.concatenate([boxes[:, 2] + 1.0, jnp.full((pad,), big + 1.0, jnp.float32)])
    y1 = jnp.concatenate([boxes[:, 3] + 1.0, jnp.full((pad,), big + 1.0, jnp.float32)])
    sc = jnp.concatenate([scores, jnp.zeros((pad,), jnp.float32)])
    decay = _decay_call(x0, y0, x1, y1, sc)
    return scores * decay[:_N]


# speedup vs baseline: 1.4314x; 1.4314x over previous
import jax
import jax.numpy as jnp
from jax import lax
from jax.experimental import pallas as pl
from jax.experimental.pallas import tpu as pltpu
from jax.experimental.pallas import tpu_sc as plsc

_N = 5000
_L = 16
_NW = 32
_RPW = 160
_NP = _NW * _RPW
_RU = 4
_THR = 0.4


def _decay_body(x0h, y0h, x1h, y1h, sch, outh,
                x0v, y0v, x1v, y1v, scv, arv, outv):
    cid = lax.axis_index("c")
    sid = lax.axis_index("s")
    wid = sid * 2 + cid
    base = wid * _RPW

    pltpu.sync_copy(x0h, x0v)
    pltpu.sync_copy(y0h, y0v)
    pltpu.sync_copy(x1h, x1v)
    pltpu.sync_copy(y1h, y1v)
    pltpu.sync_copy(sch, scv)

    lanes = lax.iota(jnp.int32, _L)
    ones = jnp.full((_L,), 1.0, jnp.float32)

    def _bfly(v, op):
        for s in (1, 2, 4, 8):
            v = op(v, v.at[lanes ^ s].get(mode="promise_in_bounds"))
        return v

    def area_chunk(k, extm):
        sl = pl.ds(k * _L, _L)
        ext = x1v[sl] - x0v[sl]
        arv[sl] = ext * (y1v[sl] - y0v[sl])
        return jnp.maximum(extm, ext)
    extm = lax.fori_loop(0, _NP // _L, area_chunk,
                         jnp.zeros((_L,), jnp.float32))
    extm = _bfly(extm, jnp.maximum)

    def rmax_chunk(k, m):
        return jnp.maximum(m, x1v[pl.ds(base + k * _L, _L)])
    rx1m = lax.fori_loop(0, _RPW // _L, rmax_chunk,
                         jnp.zeros((_L,), jnp.float32))
    hi_splat = _bfly(rx1m, jnp.maximum)
    lo_splat = _bfly(x0v[pl.ds(base, _L)], jnp.minimum) - extm

    def cnt_chunk(k, c):
        lo_c, hi_c = c
        jx0 = x0v[pl.ds(k * _L, _L)]
        one = jnp.full((_L,), 1, jnp.int32)
        zero = jnp.full((_L,), 0, jnp.int32)
        lo_c = lo_c + jnp.where(jx0 < lo_splat, one, zero)
        hi_c = hi_c + jnp.where(jx0 <= hi_splat, one, zero)
        return (lo_c, hi_c)
    zi = jnp.zeros((_L,), jnp.int32)
    lo_c, hi_c = lax.fori_loop(0, _NP // _L, cnt_chunk, (zi, zi))
    lo_idx = _bfly(lo_c, jnp.add)[0]
    hi_idx = _bfly(hi_c, jnp.add)[0]
    lo_chunk = lax.shift_right_logical(lo_idx, 4)
    hi_chunk = lax.shift_right_logical(hi_idx + 15, 4)

    def row_block(rb, carry):
        i0 = base + rb * _L
        sl_i = pl.ds(i0, _L)
        rx0 = x0v[sl_i]
        ry0 = y0v[sl_i]
        rx1 = x1v[sl_i]
        ry1 = y1v[sl_i]
        rar = arv[sl_i]
        rsc = scv[sl_i]

        res = jnp.zeros((_L,), jnp.float32)
        for g in range(_L // _RU):
            ix0 = [rx0[g * _RU + r] for r in range(_RU)]
            iy0 = [ry0[g * _RU + r] for r in range(_RU)]
            ix1 = [rx1[g * _RU + r] for r in range(_RU)]
            iy1 = [ry1[g * _RU + r] for r in range(_RU)]
            iar = [rar[g * _RU + r] for r in range(_RU)]
            isc = [rsc[g * _RU + r] for r in range(_RU)]

            def jchunk(k, accs):
                sl = pl.ds(k * _L, _L)
                jx0 = x0v[sl]
                jy0 = y0v[sl]
                jx1 = x1v[sl]
                jy1 = y1v[sl]
                js = scv[sl]
                ja = arv[sl]
                nxt = []
                for r in range(_RU):
                    wx = jnp.maximum(jnp.minimum(ix1[r], jx1) - jnp.maximum(ix0[r], jx0), 0.0)
                    wy = jnp.maximum(jnp.minimum(iy1[r], jy1) - jnp.maximum(iy0[r], jy0), 0.0)
                    inter = wx * wy
                    union = iar[r] + ja - inter
                    hit = (inter > _THR * union) & (js > isc[r])
                    ratio = (union - inter) / union
                    nxt.append(accs[r] * jnp.where(hit, ratio, 1.0))
                return tuple(nxt)

            accs = lax.fori_loop(lo_chunk, hi_chunk, jchunk, (ones,) * _RU)
            for r in range(_RU):
                a = _bfly(accs[r], jnp.multiply)
                res = jnp.where(lanes == (g * _RU + r), a[0], res)
        outv[pl.ds(rb * _L, _L)] = res
        return carry

    lax.fori_loop(0, _RPW // _L, row_block, 0)
    pltpu.sync_copy(outv, outh.at[pl.ds(base, _RPW)])


_mesh = plsc.VectorSubcoreMesh(core_axis_name="c", subcore_axis_name="s")

_decay_call = pl.kernel(
    _decay_body,
    out_type=jax.ShapeDtypeStruct((_NP,), jnp.float32),
    mesh=_mesh,
    scratch_types=[
        pltpu.VMEM((_NP,), jnp.float32),
        pltpu.VMEM((_NP,), jnp.float32),
        pltpu.VMEM((_NP,), jnp.float32),
        pltpu.VMEM((_NP,), jnp.float32),
        pltpu.VMEM((_NP,), jnp.float32),
        pltpu.VMEM((_NP,), jnp.float32),
        pltpu.VMEM((_RPW,), jnp.float32),
    ],
)


def kernel(boxes, scores):
    order = jnp.argsort(boxes[:, 0])
    bs = boxes[order]
    ss = scores[order]
    pad = _NP - _N
    big = jnp.float32(4.0e8)
    x0 = jnp.concatenate([bs[:, 0], jnp.full((pad,), big, jnp.float32)])
    y0 = jnp.concatenate([bs[:, 1], jnp.full((pad,), big, jnp.float32)])
    x1 = jnp.concatenate([bs[:, 2] + 1.0, jnp.full((pad,), big + 1.0, jnp.float32)])
    y1 = jnp.concatenate([bs[:, 3] + 1.0, jnp.full((pad,), big + 1.0, jnp.float32)])
    sc = jnp.concatenate([ss, jnp.zeros((pad,), jnp.float32)])
    decay_sorted = _decay_call(x0, y0, x1, y1, sc)
    decay = jnp.zeros((_N,), jnp.float32).at[order].set(decay_sorted[:_N])
    return scores * decay


# speedup vs baseline: 1.8625x; 1.3012x over previous
import jax
import jax.numpy as jnp
from jax import lax
from jax.experimental import pallas as pl
from jax.experimental.pallas import tpu as pltpu
from jax.experimental.pallas import tpu_sc as plsc

_N = 5000
_L = 16
_NW = 32
_RPW = 160
_NP = _NW * _RPW
_NC = _NP // _L
_NB = _RPW // _L
_RU = 4
_THR = 0.4


def _decay_body(sx0h, sidxh, y0h, x1h, y1h, sch, outh,
                sx0v, sidxv, y0v, x1v, y1v, scv,
                gy0v, gx1v, gy1v, gscv, arv, outv, sem):
    cid = lax.axis_index("c")
    sid = lax.axis_index("s")
    wid = sid * 2 + cid

    pltpu.sync_copy(sx0h, sx0v)
    pltpu.sync_copy(sidxh, sidxv)
    pltpu.sync_copy(y0h, y0v)
    pltpu.sync_copy(x1h, x1v)
    pltpu.sync_copy(y1h, y1v)
    pltpu.sync_copy(sch, scv)

    lanes = lax.iota(jnp.int32, _L)
    ones = jnp.full((_L,), 1.0, jnp.float32)

    def _bfly(v, op):
        for s in (1, 2, 4, 8):
            v = op(v, v.at[lanes ^ s].get(mode="promise_in_bounds"))
        return v

    def perm_chunk(k, extm):
        sl = pl.ds(k * _L, _L)
        idx = sidxv[sl]
        gy0 = plsc.load_gather(y0v, [idx])
        gx1 = plsc.load_gather(x1v, [idx])
        gy1 = plsc.load_gather(y1v, [idx])
        gsc = plsc.load_gather(scv, [idx])
        gy0v[sl] = gy0
        gx1v[sl] = gx1
        gy1v[sl] = gy1
        gscv[sl] = gsc
        ext = gx1 - sx0v[sl]
        arv[sl] = ext * (gy1 - gy0)
        return jnp.maximum(extm, ext)
    extm = lax.fori_loop(0, _NC, perm_chunk, jnp.zeros((_L,), jnp.float32))
    extm = _bfly(extm, jnp.maximum)

    def _bisect(pred):
        def step(_, lohi):
            lo, hi = lohi
            mid = lax.shift_right_logical(lo + hi, 1)
            p = pred(mid)
            lo = jnp.where(p, lo, mid + 1)
            hi = jnp.where(p, mid, hi)
            return (lo, hi)
        lo, _ = lax.fori_loop(0, 9, step, (jnp.int32(0), jnp.int32(_NC)))
        return lo

    def row_block(rb, carry):
        blk = wid + _NW * rb
        i0 = blk * _L
        sl_i = pl.ds(i0, _L)
        rx0 = sx0v[sl_i]
        ry0 = gy0v[sl_i]
        rx1 = gx1v[sl_i]
        ry1 = gy1v[sl_i]
        rar = arv[sl_i]
        rsc = gscv[sl_i]

        lo_spl = _bfly(rx0, jnp.minimum) - extm
        hi_spl = _bfly(rx1, jnp.maximum)
        zi = jnp.zeros((_L,), jnp.int32)
        oi = jnp.full((_L,), 1, jnp.int32)

        def lo_pred(c):
            v = sx0v[pl.ds(c * _L, _L)]
            return jnp.where(v >= lo_spl, oi, zi)[_L - 1] > 0
        def hi_pred(c):
            v = sx0v[pl.ds(c * _L, _L)]
            return jnp.where(v > hi_spl, oi, zi)[0] > 0
        lo_chunk = _bisect(lo_pred)
        hi_chunk = _bisect(hi_pred)

        res = jnp.zeros((_L,), jnp.float32)
        for g in range(_L // _RU):
            ix0 = [rx0[g * _RU + r] for r in range(_RU)]
            iy0 = [ry0[g * _RU + r] for r in range(_RU)]
            ix1 = [rx1[g * _RU + r] for r in range(_RU)]
            iy1 = [ry1[g * _RU + r] for r in range(_RU)]
            iar = [rar[g * _RU + r] for r in range(_RU)]
            isc = [rsc[g * _RU + r] for r in range(_RU)]

            def jchunk(k, accs):
                sl = pl.ds(k * _L, _L)
                jx0 = sx0v[sl]
                jy0 = gy0v[sl]
                jx1 = gx1v[sl]
                jy1 = gy1v[sl]
                js = gscv[sl]
                ja = arv[sl]
                nxt = []
                for r in range(_RU):
                    wx = jnp.maximum(jnp.minimum(ix1[r], jx1) - jnp.maximum(ix0[r], jx0), 0.0)
                    wy = jnp.maximum(jnp.minimum(iy1[r], jy1) - jnp.maximum(iy0[r], jy0), 0.0)
                    inter = wx * wy
                    union = iar[r] + ja - inter
                    hit = (inter > _THR * union) & (js > isc[r])
                    ratio = (union - inter) / union
                    nxt.append(accs[r] * jnp.where(hit, ratio, 1.0))
                return tuple(nxt)

            accs = lax.fori_loop(lo_chunk, hi_chunk, jchunk, (ones,) * _RU)
            for r in range(_RU):
                a = _bfly(accs[r], jnp.multiply)
                res = jnp.where(lanes == (g * _RU + r), a[0], res)
        outv[pl.ds(rb * _L, _L)] = res
        pltpu.async_copy(outv.at[pl.ds(rb * _L, _L)],
                         outh.at[pl.ds(i0, _L)], sem)
        return carry

    lax.fori_loop(0, _NB, row_block, 0)
    def drain(rb, carry):
        pltpu.make_async_copy(outv.at[pl.ds(0, _L)],
                              outh.at[pl.ds(0, _L)], sem).wait()
        return carry
    lax.fori_loop(0, _NB, drain, 0)


_mesh = plsc.VectorSubcoreMesh(core_axis_name="c", subcore_axis_name="s")

_decay_call = pl.kernel(
    _decay_body,
    out_type=jax.ShapeDtypeStruct((_NP,), jnp.float32),
    mesh=_mesh,
    scratch_types=[
        pltpu.VMEM((_NP,), jnp.float32),
        pltpu.VMEM((_NP,), jnp.int32),
        pltpu.VMEM((_NP,), jnp.float32),
        pltpu.VMEM((_NP,), jnp.float32),
        pltpu.VMEM((_NP,), jnp.float32),
        pltpu.VMEM((_NP,), jnp.float32),
        pltpu.VMEM((_NP,), jnp.float32),
        pltpu.VMEM((_NP,), jnp.float32),
        pltpu.VMEM((_NP,), jnp.float32),
        pltpu.VMEM((_NP,), jnp.float32),
        pltpu.VMEM((_NP,), jnp.float32),
        pltpu.VMEM((_RPW,), jnp.float32),
        pltpu.SemaphoreType.DMA,
    ],
    compiler_params=pltpu.CompilerParams(needs_layout_passes=False),
)


def kernel(boxes, scores):
    idx = lax.iota(jnp.int32, _N)
    sx0, sidx = lax.sort((boxes[:, 0], idx), num_keys=1, is_stable=False)
    pad = _NP - _N
    big = jnp.float32(4.0e8)
    padi = lax.iota(jnp.int32, pad) + _N
    sx0p = jnp.concatenate([sx0, jnp.full((pad,), big, jnp.float32)])
    sidxp = jnp.concatenate([sidx, padi])
    y0 = jnp.concatenate([boxes[:, 1], jnp.full((pad,), big, jnp.float32)])
    x1 = jnp.concatenate([boxes[:, 2] + 1.0, jnp.full((pad,), big + 1.0, jnp.float32)])
    y1 = jnp.concatenate([boxes[:, 3] + 1.0, jnp.full((pad,), big + 1.0, jnp.float32)])
    sc = jnp.concatenate([scores, jnp.zeros((pad,), jnp.float32)])
    decay_sorted = _decay_call(sx0p, sidxp, y0, x1, y1, sc)
    decay = jnp.zeros((_N,), jnp.float32).at[sidx].set(decay_sorted[:_N])
    return scores * decay
